# baseline (speedup 1.0000x reference)
import functools

import jax
import jax.numpy as jnp
from jax import lax
from jax.experimental import pallas as pl
from jax.experimental.pallas import tpu as pltpu
from jax.experimental.pallas import tpu_sc as plsc

NC, NS = 2, 16
NW = NC * NS
BL = 128

N = 10000
E = 320000
M = 2500
G = 256
D = 128
F = 9
V = 128

N_PAD = 10240
NH = N_PAD // 2
NH_PAD = 5248
M_PAD = 2560
G_PAD = 384
H3_PAD = 12288
E_PAD = 327680
DEG_PAD = 655360
POOL_PAD = 32768


def _mesh():
    return plsc.VectorSubcoreMesh(
        core_axis_name="c", subcore_axis_name="s",
        num_cores=NC, num_subcores=NS)


@functools.lru_cache(None)
def _sc_pair_split(rpt: int, a_rows: int):
    ar16 = a_rows // NS

    def body(gidx, sidx, table, zeros, out, gv, sv, b0, acc):
        c = lax.axis_index("c")
        s = lax.axis_index("s")
        row0 = (c * NS + s) * rpt
        pltpu.sync_copy(zeros, acc.at[pl.ds(s * ar16, ar16)])
        pltpu.sync_copy(gidx.at[pl.ds(row0, rpt)], gv)
        pltpu.sync_copy(sidx.at[pl.ds(row0, rpt)], sv)
        plsc.subcore_barrier()

        def step(j, carry):
            pltpu.sync_copy(table.at[gv.at[j]], b0)
            pltpu.sync_copy(b0, acc.at[sv.at[j]], add=True)
            return carry

        lax.fori_loop(0, rpt, step, 0)
        plsc.subcore_barrier()
        pltpu.sync_copy(acc.at[pl.ds(s * ar16, ar16)],
                        out.at[c, pl.ds(s * ar16, ar16)])

    return pl.kernel(
        body,
        out_type=jax.ShapeDtypeStruct((NC, a_rows, D), jnp.float32),
        mesh=_mesh(),
        scratch_types=[
            pltpu.VMEM((rpt, BL), jnp.int32),
            pltpu.VMEM((rpt, BL), jnp.int32),
            pltpu.VMEM((BL, D), jnp.float32),
            pltpu.VMEM_SHARED((a_rows, D), jnp.float32),
        ],
    )


@functools.lru_cache(None)
def _sc_range_split(rpt: int, t_rows: int):
    ar16 = NH_PAD // NS

    def body(gidx, sidx, table, zeros, out, gv, sv, b0, acc):
        c = lax.axis_index("c")
        s = lax.axis_index("s")
        row0 = s * rpt
        pltpu.sync_copy(zeros, acc.at[pl.ds(s * ar16, ar16)])
        pltpu.sync_copy(gidx.at[pl.ds(row0, rpt)], gv)
        pltpu.sync_copy(sidx.at[c, pl.ds(row0, rpt)], sv)
        plsc.subcore_barrier()

        def step(j, carry):
            pltpu.sync_copy(table.at[gv.at[j]], b0)
            pltpu.sync_copy(b0, acc.at[sv.at[j]], add=True)
            return carry

        lax.fori_loop(0, rpt, step, 0)
        plsc.subcore_barrier()
        pltpu.sync_copy(acc.at[pl.ds(s * ar16, ar16)],
                        out.at[c, pl.ds(s * ar16, ar16)])

    return pl.kernel(
        body,
        out_type=jax.ShapeDtypeStruct((NC, NH_PAD, D), jnp.float32),
        mesh=_mesh(),
        scratch_types=[
            pltpu.VMEM((rpt, BL), jnp.int32),
            pltpu.VMEM((rpt, BL), jnp.int32),
            pltpu.VMEM((BL, D), jnp.float32),
            pltpu.VMEM_SHARED((NH_PAD, D), jnp.float32),
        ],
    )


def _vjoin(hs_ref):
    return jnp.concatenate([hs_ref[0, :NH], hs_ref[1, :NH]], axis=0)


def _tc_enc_hv(xt, at, wv, bv):
    def body(x_ref, at_ref, w_ref, b_ref, o_ref):
        h = jnp.zeros((N_PAD, D), jnp.float32)
        for f in range(F):
            oh = (jax.lax.broadcasted_iota(jnp.int32, (V, N_PAD), 0)
                  == x_ref[f][None, :]).astype(jnp.float32)
            h = h + jax.lax.dot_general(
                oh, at_ref[f], (((0,), (0,)), ((), ())),
                preferred_element_type=jnp.float32)
        o_ref[...] = (jnp.dot(h, w_ref[...], preferred_element_type=jnp.float32)
                      + b_ref[...])
    return pl.pallas_call(
        body, out_shape=jax.ShapeDtypeStruct((N_PAD, D), jnp.float32))(
            xt, at, wv, bv)


def _tc_hv(hs, cnt, wv, bv):
    def body(hs_ref, c_ref, w_ref, b_ref, o_ref):
        dv = jnp.maximum(c_ref[0] + c_ref[1], 1.0)
        h = jnp.maximum(_vjoin(hs_ref) / dv, 0.0)
        o_ref[...] = (jnp.dot(h, w_ref[...], preferred_element_type=jnp.float32)
                      + b_ref[...])
    return pl.pallas_call(
        body, out_shape=jax.ShapeDtypeStruct((N_PAD, D), jnp.float32))(hs, cnt, wv, bv)


def _tc_he(hs, cnt, we, be):
    def body(hs_ref, c_ref, w_ref, b_ref, o_ref):
        de = jnp.maximum(c_ref[0] + c_ref[1], 1.0)
        he = jnp.maximum((hs_ref[0] + hs_ref[1]) / de, 0.0)
        o_ref[...] = (jnp.dot(he, w_ref[...], preferred_element_type=jnp.float32)
                      + b_ref[...])
    return pl.pallas_call(
        body, out_shape=jax.ShapeDtypeStruct((M_PAD, D), jnp.float32))(hs, cnt, we, be)


def _tc_h3(hs, cnt):
    def body(hs_ref, c_ref, o_ref):
        dv = jnp.maximum(c_ref[0] + c_ref[1], 1.0)
        h3 = jnp.maximum(_vjoin(hs_ref) / dv, 0.0)
        o_ref[pl.ds(0, N_PAD), :] = h3
        o_ref[pl.ds(N_PAD, H3_PAD - N_PAD), :] = jnp.zeros(
            (H3_PAD - N_PAD, D), jnp.float32)
    return pl.pallas_call(
        body, out_shape=jax.ShapeDtypeStruct((H3_PAD, D), jnp.float32))(hs, cnt)


def _tc_mlp(gs, w1, b1, w2, b2, w3, b3):
    def body(gs_ref, w1r, b1r, w2r, b2r, w3r, b3r, o_ref):
        g = (gs_ref[0] + gs_ref[1])[:G]
        g = jnp.maximum(jnp.dot(g, w1r[...], preferred_element_type=jnp.float32)
                        + b1r[...], 0.0)
        g = jnp.maximum(jnp.dot(g, w2r[...], preferred_element_type=jnp.float32)
                        + b2r[...], 0.0)
        o_ref[...] = (jnp.dot(g, w3r[...], preferred_element_type=jnp.float32)
                      + b3r[...])
    return pl.pallas_call(
        body, out_shape=jax.ShapeDtypeStruct((G, 1), jnp.float32))(
            gs, w1, b1, w2, b2, w3, b3)


def _range_sidx(tgt):
    c0 = jnp.where(tgt < NH, tgt, NH_PAD - 1)
    c1 = jnp.where(tgt >= NH, tgt - NH, NH_PAD - 1)
    return jnp.stack([c0, c1]).reshape(2, -1, BL)


def kernel(x, v_idx, e_idx, batch, atom_tables,
           layer_Wv, layer_bv, layer_We, layer_be,
           mlp_W1, mlp_b1, mlp_W2, mlp_b2, mlp_W3, mlp_b3):
    i32 = jnp.int32
    f32 = jnp.float32
    v_idx = v_idx.astype(i32)
    e_idx = e_idx.astype(i32)

    xt = jnp.concatenate(
        [x.astype(i32), jnp.zeros((N_PAD - N, F), i32)]).T

    pad_e = E_PAD - E
    vidx_g = jnp.concatenate([v_idx, jnp.zeros((pad_e,), i32)]).reshape(-1, BL)
    eidx_s = jnp.concatenate(
        [e_idx, jnp.full((pad_e,), M_PAD - 1, i32)]).reshape(-1, BL)
    eidx_g = jnp.concatenate([e_idx, jnp.zeros((pad_e,), i32)]).reshape(-1, BL)
    vidx_s = _range_sidx(jnp.concatenate(
        [v_idx, jnp.full((pad_e,), N_PAD - 1, i32)]))

    deg_rows = M_PAD + N_PAD
    deg_t = jnp.concatenate(
        [e_idx, M_PAD + v_idx,
         jnp.full((DEG_PAD - 2 * E,), deg_rows - 1, i32)])
    deg_g = (deg_t % 8).reshape(-1, BL)
    deg_s = (deg_t // 8).reshape(-1, BL)
    lane_pat = (jnp.arange(BL, dtype=i32)[None, :] // 16
                == jnp.arange(8, dtype=i32)[:, None]).astype(f32)

    pool_g = jnp.concatenate(
        [jnp.arange(N_PAD, dtype=i32),
         jnp.zeros((POOL_PAD - N_PAD,), i32)]).reshape(-1, BL)
    pool_s = jnp.concatenate(
        [batch.astype(i32), jnp.full((N_PAD - N,), G_PAD - 1, i32),
         jnp.full((POOL_PAD - N_PAD,), G_PAD - 1, i32)]).reshape(-1, BL)

    zeros128 = jnp.zeros((NH_PAD // NS, D), f32)

    bv = layer_bv.reshape(3, 1, D)
    be = layer_be.reshape(3, 1, D)
    b1 = mlp_b1.reshape(1, D)
    b2 = mlp_b2.reshape(1, D)
    b3 = mlp_b3.reshape(1, 1)

    deg = _sc_pair_split(DEG_PAD // NW // BL, 1664)(
        deg_g, deg_s, lane_pat, zeros128[:104])
    dcnt = deg[:, :deg_rows // 8].reshape(NC, deg_rows // 8, 8, 16)[..., 0]
    dcnt = dcnt.reshape(NC, deg_rows)
    cnt_e = dcnt[:, :M_PAD, None]
    cnt_v = dcnt[:, M_PAD:, None]

    hv = _tc_enc_hv(xt, atom_tables, layer_Wv[0], bv[0])
    hs = None
    for l in range(3):
        hes = _sc_pair_split(E_PAD // NW // BL, M_PAD)(
            vidx_g, eidx_s, hv, zeros128[:M_PAD // NS])
        he = _tc_he(hes, cnt_e, layer_We[l], be[l])
        hs = _sc_range_split(E_PAD // NS // BL, M_PAD)(
            eidx_g, vidx_s, he, zeros128)
        if l < 2:
            hv = _tc_hv(hs, cnt_v, layer_Wv[l + 1], bv[l + 1])

    h3 = _tc_h3(hs, cnt_v)
    gs = _sc_pair_split(POOL_PAD // NW // BL, G_PAD)(
        pool_g, pool_s, h3, zeros128[:G_PAD // NS])
    out = _tc_mlp(gs, mlp_W1, b1, mlp_W2, b2, mlp_W3, b3)
    return out.reshape(-1)


# speedup vs baseline: 1.0438x; 1.0438x over previous
import functools

import jax
import jax.numpy as jnp
from jax import lax
from jax.experimental import pallas as pl
from jax.experimental.pallas import tpu as pltpu
from jax.experimental.pallas import tpu_sc as plsc

NC, NS = 2, 16
NW = NC * NS
BL = 128

N = 10000
E = 320000
M = 2500
G = 256
D = 128
F = 9
V = 128

N_PAD = 10240
NH = N_PAD // 2
NH_PAD = 5248
M_PAD = 2560
G_PAD = 384
H3_PAD = 12288
E_PAD = 327680
DEG_PAD = 655360
POOL_PAD = 32768


def _mesh():
    return plsc.VectorSubcoreMesh(
        core_axis_name="c", subcore_axis_name="s",
        num_cores=NC, num_subcores=NS)


@functools.lru_cache(None)
def _sc_pair_split(rpt: int, a_rows: int):
    ar16 = a_rows // NS

    def body(gidx, sidx, table, zeros, out, gv, sv, b0, b1, acc, sem0, sem1):
        c = lax.axis_index("c")
        s = lax.axis_index("s")
        row0 = (c * NS + s) * rpt
        pltpu.sync_copy(zeros, acc.at[pl.ds(s * ar16, ar16)])
        pltpu.sync_copy(gidx.at[pl.ds(row0, rpt)], gv)
        pltpu.sync_copy(sidx.at[pl.ds(row0, rpt)], sv)
        plsc.subcore_barrier()

        def start(j, buf, sem):
            pltpu.async_copy(table.at[gv.at[j]], buf, sem)

        def finish(j, buf, sem):
            pltpu.make_async_copy(table.at[gv.at[j]], buf, sem).wait()
            pltpu.sync_copy(buf, acc.at[sv.at[j]], add=True)

        start(0, b0, sem0)
        start(1, b1, sem1)

        def step(i, carry):
            j0 = 2 * i

            finish(j0, b0, sem0)

            @pl.when(j0 + 2 < rpt)
            def _():
                start(j0 + 2, b0, sem0)

            finish(j0 + 1, b1, sem1)

            @pl.when(j0 + 3 < rpt)
            def _():
                start(j0 + 3, b1, sem1)

            return carry

        lax.fori_loop(0, rpt // 2, step, 0)
        plsc.subcore_barrier()
        pltpu.sync_copy(acc.at[pl.ds(s * ar16, ar16)],
                        out.at[c, pl.ds(s * ar16, ar16)])

    return pl.kernel(
        body,
        out_type=jax.ShapeDtypeStruct((NC, a_rows, D), jnp.float32),
        mesh=_mesh(),
        scratch_types=[
            pltpu.VMEM((rpt, BL), jnp.int32),
            pltpu.VMEM((rpt, BL), jnp.int32),
            pltpu.VMEM((BL, D), jnp.float32),
            pltpu.VMEM((BL, D), jnp.float32),
            pltpu.VMEM_SHARED((a_rows, D), jnp.float32),
            pltpu.SemaphoreType.DMA,
            pltpu.SemaphoreType.DMA,
        ],
    )


@functools.lru_cache(None)
def _sc_range_split(rpt: int, t_rows: int):
    ar16 = NH_PAD // NS

    def body(gidx, sidx, table, zeros, out, gv, sv, b0, b1, acc, sem0, sem1):
        c = lax.axis_index("c")
        s = lax.axis_index("s")
        row0 = s * rpt
        pltpu.sync_copy(zeros, acc.at[pl.ds(s * ar16, ar16)])
        pltpu.sync_copy(gidx.at[pl.ds(row0, rpt)], gv)
        pltpu.sync_copy(sidx.at[c, pl.ds(row0, rpt)], sv)
        plsc.subcore_barrier()

        def start(j, buf, sem):
            pltpu.async_copy(table.at[gv.at[j]], buf, sem)

        def finish(j, buf, sem):
            pltpu.make_async_copy(table.at[gv.at[j]], buf, sem).wait()
            pltpu.sync_copy(buf, acc.at[sv.at[j]], add=True)

        start(0, b0, sem0)
        start(1, b1, sem1)

        def step(i, carry):
            j0 = 2 * i

            finish(j0, b0, sem0)

            @pl.when(j0 + 2 < rpt)
            def _():
                start(j0 + 2, b0, sem0)

            finish(j0 + 1, b1, sem1)

            @pl.when(j0 + 3 < rpt)
            def _():
                start(j0 + 3, b1, sem1)

            return carry

        lax.fori_loop(0, rpt // 2, step, 0)
        plsc.subcore_barrier()
        pltpu.sync_copy(acc.at[pl.ds(s * ar16, ar16)],
                        out.at[c, pl.ds(s * ar16, ar16)])

    return pl.kernel(
        body,
        out_type=jax.ShapeDtypeStruct((NC, NH_PAD, D), jnp.float32),
        mesh=_mesh(),
        scratch_types=[
            pltpu.VMEM((rpt, BL), jnp.int32),
            pltpu.VMEM((rpt, BL), jnp.int32),
            pltpu.VMEM((BL, D), jnp.float32),
            pltpu.VMEM((BL, D), jnp.float32),
            pltpu.VMEM_SHARED((NH_PAD, D), jnp.float32),
            pltpu.SemaphoreType.DMA,
            pltpu.SemaphoreType.DMA,
        ],
    )


def _vjoin(hs_ref):
    return jnp.concatenate([hs_ref[0, :NH], hs_ref[1, :NH]], axis=0)


def _tc_enc_hv(xt, at, wv, bv):
    def body(x_ref, at_ref, w_ref, b_ref, o_ref):
        h = jnp.zeros((N_PAD, D), jnp.float32)
        for f in range(F):
            oh = (jax.lax.broadcasted_iota(jnp.int32, (V, N_PAD), 0)
                  == x_ref[f][None, :]).astype(jnp.float32)
            h = h + jax.lax.dot_general(
                oh, at_ref[f], (((0,), (0,)), ((), ())),
                preferred_element_type=jnp.float32)
        o_ref[...] = (jnp.dot(h, w_ref[...], preferred_element_type=jnp.float32,
                precision=jax.lax.Precision.HIGHEST)
                      + b_ref[...])
    return pl.pallas_call(
        body, out_shape=jax.ShapeDtypeStruct((N_PAD, D), jnp.float32))(
            xt, at, wv, bv)


def _tc_hv(hs, cnt, wv, bv):
    def body(hs_ref, c_ref, w_ref, b_ref, o_ref):
        dv = jnp.maximum(c_ref[0] + c_ref[1], 1.0)
        h = jnp.maximum(_vjoin(hs_ref) / dv, 0.0)
        o_ref[...] = (jnp.dot(h, w_ref[...], preferred_element_type=jnp.float32,
                precision=jax.lax.Precision.HIGHEST)
                      + b_ref[...])
    return pl.pallas_call(
        body, out_shape=jax.ShapeDtypeStruct((N_PAD, D), jnp.float32))(hs, cnt, wv, bv)


def _tc_he(hs, cnt, we, be):
    def body(hs_ref, c_ref, w_ref, b_ref, o_ref):
        de = jnp.maximum(c_ref[0] + c_ref[1], 1.0)
        he = jnp.maximum((hs_ref[0] + hs_ref[1]) / de, 0.0)
        o_ref[...] = (jnp.dot(he, w_ref[...], preferred_element_type=jnp.float32,
                precision=jax.lax.Precision.HIGHEST)
                      + b_ref[...])
    return pl.pallas_call(
        body, out_shape=jax.ShapeDtypeStruct((M_PAD, D), jnp.float32))(hs, cnt, we, be)


def _tc_h3(hs, cnt):
    def body(hs_ref, c_ref, o_ref):
        dv = jnp.maximum(c_ref[0] + c_ref[1], 1.0)
        h3 = jnp.maximum(_vjoin(hs_ref) / dv, 0.0)
        o_ref[pl.ds(0, N_PAD), :] = h3
        o_ref[pl.ds(N_PAD, H3_PAD - N_PAD), :] = jnp.zeros(
            (H3_PAD - N_PAD, D), jnp.float32)
    return pl.pallas_call(
        body, out_shape=jax.ShapeDtypeStruct((H3_PAD, D), jnp.float32))(hs, cnt)


def _tc_mlp(gs, w1, b1, w2, b2, w3, b3):
    def body(gs_ref, w1r, b1r, w2r, b2r, w3r, b3r, o_ref):
        g = (gs_ref[0] + gs_ref[1])[:G]
        g = jnp.maximum(jnp.dot(g, w1r[...], preferred_element_type=jnp.float32,
                precision=jax.lax.Precision.HIGHEST)
                        + b1r[...], 0.0)
        g = jnp.maximum(jnp.dot(g, w2r[...], preferred_element_type=jnp.float32,
                precision=jax.lax.Precision.HIGHEST)
                        + b2r[...], 0.0)
        o_ref[...] = (jnp.dot(g, w3r[...], preferred_element_type=jnp.float32,
                precision=jax.lax.Precision.HIGHEST)
                      + b3r[...])
    return pl.pallas_call(
        body, out_shape=jax.ShapeDtypeStruct((G, 1), jnp.float32))(
            gs, w1, b1, w2, b2, w3, b3)


def _range_sidx(tgt):
    c0 = jnp.where(tgt < NH, tgt, NH_PAD - 1)
    c1 = jnp.where(tgt >= NH, tgt - NH, NH_PAD - 1)
    return jnp.stack([c0, c1]).reshape(2, -1, BL)


def kernel(x, v_idx, e_idx, batch, atom_tables,
           layer_Wv, layer_bv, layer_We, layer_be,
           mlp_W1, mlp_b1, mlp_W2, mlp_b2, mlp_W3, mlp_b3):
    i32 = jnp.int32
    f32 = jnp.float32
    v_idx = v_idx.astype(i32)
    e_idx = e_idx.astype(i32)

    xt = jnp.concatenate(
        [x.astype(i32), jnp.zeros((N_PAD - N, F), i32)]).T

    pad_e = E_PAD - E
    vidx_g = jnp.concatenate([v_idx, jnp.zeros((pad_e,), i32)]).reshape(-1, BL)
    eidx_s = jnp.concatenate(
        [e_idx, jnp.full((pad_e,), M_PAD - 1, i32)]).reshape(-1, BL)
    eidx_g = jnp.concatenate([e_idx, jnp.zeros((pad_e,), i32)]).reshape(-1, BL)
    vidx_s = _range_sidx(jnp.concatenate(
        [v_idx, jnp.full((pad_e,), N_PAD - 1, i32)]))

    deg_rows = M_PAD + N_PAD
    deg_t = jnp.concatenate(
        [e_idx, M_PAD + v_idx,
         jnp.full((DEG_PAD - 2 * E,), deg_rows - 1, i32)])
    deg_g = (deg_t % 8).reshape(-1, BL)
    deg_s = (deg_t // 8).reshape(-1, BL)
    lane_pat = (jnp.arange(BL, dtype=i32)[None, :] // 16
                == jnp.arange(8, dtype=i32)[:, None]).astype(f32)

    pool_g = jnp.concatenate(
        [jnp.arange(N_PAD, dtype=i32),
         jnp.zeros((POOL_PAD - N_PAD,), i32)]).reshape(-1, BL)
    pool_s = jnp.concatenate(
        [batch.astype(i32), jnp.full((N_PAD - N,), G_PAD - 1, i32),
         jnp.full((POOL_PAD - N_PAD,), G_PAD - 1, i32)]).reshape(-1, BL)

    zeros128 = jnp.zeros((NH_PAD // NS, D), f32)

    bv = layer_bv.reshape(3, 1, D)
    be = layer_be.reshape(3, 1, D)
    b1 = mlp_b1.reshape(1, D)
    b2 = mlp_b2.reshape(1, D)
    b3 = mlp_b3.reshape(1, 1)

    deg = _sc_pair_split(DEG_PAD // NW // BL, 1664)(
        deg_g, deg_s, lane_pat, zeros128[:104])
    dcnt = deg[:, :deg_rows // 8].reshape(NC, deg_rows // 8, 8, 16)[..., 0]
    dcnt = dcnt.reshape(NC, deg_rows)
    cnt_e = dcnt[:, :M_PAD, None]
    cnt_v = dcnt[:, M_PAD:, None]

    hv = _tc_enc_hv(xt, atom_tables, layer_Wv[0], bv[0])
    hs = None
    for l in range(3):
        hes = _sc_pair_split(E_PAD // NW // BL, M_PAD)(
            vidx_g, eidx_s, hv, zeros128[:M_PAD // NS])
        he = _tc_he(hes, cnt_e, layer_We[l], be[l])
        hs = _sc_range_split(E_PAD // NS // BL, M_PAD)(
            eidx_g, vidx_s, he, zeros128)
        if l < 2:
            hv = _tc_hv(hs, cnt_v, layer_Wv[l + 1], bv[l + 1])

    h3 = _tc_h3(hs, cnt_v)
    gs = _sc_pair_split(POOL_PAD // NW // BL, G_PAD)(
        pool_g, pool_s, h3, zeros128[:G_PAD // NS])
    out = _tc_mlp(gs, mlp_W1, b1, mlp_W2, b2, mlp_W3, b3)
    return out.reshape(-1)


# speedup vs baseline: 1.5484x; 1.4834x over previous
import functools

import jax
import jax.numpy as jnp
from jax import lax
from jax.experimental import pallas as pl
from jax.experimental.pallas import tpu as pltpu
from jax.experimental.pallas import tpu_sc as plsc

NC, NS = 2, 16
NW = NC * NS
BL = 128

N = 10000
E = 320000
M = 2500
G = 256
D = 128
F = 9
V = 128

N_PAD = 10240
NH = N_PAD // 2
NH_PAD = 5248
M_PAD = 2560
G_PAD = 384
H3_PAD = 12288
E_PAD = 327680
DEG_PAD = 655360
POOL_PAD = 32768


def _mesh():
    return plsc.VectorSubcoreMesh(
        core_axis_name="c", subcore_axis_name="s",
        num_cores=NC, num_subcores=NS)


@functools.lru_cache(None)
def _sc_pair_split(rpt: int, a_rows: int):
    ar16 = a_rows // NS

    def body(gidx, sidx, table, zeros, out, gv, sv, b0, b1, acc, sem0, sem1):
        c = lax.axis_index("c")
        s = lax.axis_index("s")
        row0 = (c * NS + s) * rpt
        pltpu.sync_copy(zeros, acc.at[pl.ds(s * ar16, ar16)])
        pltpu.sync_copy(gidx.at[pl.ds(row0, rpt)], gv)
        pltpu.sync_copy(sidx.at[pl.ds(row0, rpt)], sv)
        plsc.subcore_barrier()

        def start(j, buf, sem):
            pltpu.async_copy(table.at[gv.at[j]], buf, sem)

        def finish(j, buf, sem):
            pltpu.make_async_copy(table.at[gv.at[j]], buf, sem).wait()
            pltpu.sync_copy(buf, acc.at[sv.at[j]], add=True)

        start(0, b0, sem0)
        start(1, b1, sem1)

        def step(i, carry):
            j0 = 2 * i

            finish(j0, b0, sem0)

            @pl.when(j0 + 2 < rpt)
            def _():
                start(j0 + 2, b0, sem0)

            finish(j0 + 1, b1, sem1)

            @pl.when(j0 + 3 < rpt)
            def _():
                start(j0 + 3, b1, sem1)

            return carry

        lax.fori_loop(0, rpt // 2, step, 0)
        plsc.subcore_barrier()
        pltpu.sync_copy(acc.at[pl.ds(s * ar16, ar16)],
                        out.at[c, pl.ds(s * ar16, ar16)])

    return pl.kernel(
        body,
        out_type=jax.ShapeDtypeStruct((NC, a_rows, D), jnp.float32),
        mesh=_mesh(),
        scratch_types=[
            pltpu.VMEM((rpt, BL), jnp.int32),
            pltpu.VMEM((rpt, BL), jnp.int32),
            pltpu.VMEM((BL, D), jnp.float32),
            pltpu.VMEM((BL, D), jnp.float32),
            pltpu.VMEM_SHARED((a_rows, D), jnp.float32),
            pltpu.SemaphoreType.DMA,
            pltpu.SemaphoreType.DMA,
        ],
    )


@functools.lru_cache(None)
def _sc_range_split(rpt: int, t_rows: int):
    ar16 = NH_PAD // NS

    def body(gidx, sidx, table, zeros, out, gv, sv, b0, b1, acc, sem0, sem1):
        c = lax.axis_index("c")
        s = lax.axis_index("s")
        row0 = s * rpt
        pltpu.sync_copy(zeros, acc.at[pl.ds(s * ar16, ar16)])
        pltpu.sync_copy(gidx.at[pl.ds(row0, rpt)], gv)
        pltpu.sync_copy(sidx.at[c, pl.ds(row0, rpt)], sv)
        plsc.subcore_barrier()

        def start(j, buf, sem):
            pltpu.async_copy(table.at[gv.at[j]], buf, sem)

        def finish(j, buf, sem):
            pltpu.make_async_copy(table.at[gv.at[j]], buf, sem).wait()
            pltpu.sync_copy(buf, acc.at[sv.at[j]], add=True)

        start(0, b0, sem0)
        start(1, b1, sem1)

        def step(i, carry):
            j0 = 2 * i

            finish(j0, b0, sem0)

            @pl.when(j0 + 2 < rpt)
            def _():
                start(j0 + 2, b0, sem0)

            finish(j0 + 1, b1, sem1)

            @pl.when(j0 + 3 < rpt)
            def _():
                start(j0 + 3, b1, sem1)

            return carry

        lax.fori_loop(0, rpt // 2, step, 0)
        plsc.subcore_barrier()
        pltpu.sync_copy(acc.at[pl.ds(s * ar16, ar16)],
                        out.at[c, pl.ds(s * ar16, ar16)])

    return pl.kernel(
        body,
        out_type=jax.ShapeDtypeStruct((NC, NH_PAD, D), jnp.float32),
        mesh=_mesh(),
        scratch_types=[
            pltpu.VMEM((rpt, BL), jnp.int32),
            pltpu.VMEM((rpt, BL), jnp.int32),
            pltpu.VMEM((BL, D), jnp.float32),
            pltpu.VMEM((BL, D), jnp.float32),
            pltpu.VMEM_SHARED((NH_PAD, D), jnp.float32),
            pltpu.SemaphoreType.DMA,
            pltpu.SemaphoreType.DMA,
        ],
    )


def _vjoin(hs_ref):
    return jnp.concatenate([hs_ref[0, :NH], hs_ref[1, :NH]], axis=0)


def _tc_enc_hv(xt, at, wv, bv):
    def body(x_ref, at_ref, w_ref, b_ref, o_ref):
        h = jnp.zeros((N_PAD, D), jnp.float32)
        for f in range(F):
            oh = (jax.lax.broadcasted_iota(jnp.int32, (V, N_PAD), 0)
                  == x_ref[f][None, :]).astype(jnp.float32)
            h = h + jax.lax.dot_general(
                oh, at_ref[f], (((0,), (0,)), ((), ())),
                preferred_element_type=jnp.float32)
        o_ref[...] = (jnp.dot(h, w_ref[...], preferred_element_type=jnp.float32,
                precision=jax.lax.Precision.HIGHEST)
                      + b_ref[...])
    return pl.pallas_call(
        body, out_shape=jax.ShapeDtypeStruct((N_PAD, D), jnp.float32))(
            xt, at, wv, bv)


def _tc_hv(hs, cnt, wv, bv):
    def body(hs_ref, c_ref, w_ref, b_ref, o_ref):
        dv = jnp.maximum(jnp.sum(c_ref[...], axis=0), 1.0)[:, None]
        h = jnp.maximum(_vjoin(hs_ref) / dv, 0.0)
        o_ref[...] = (jnp.dot(h, w_ref[...], preferred_element_type=jnp.float32,
                precision=jax.lax.Precision.HIGHEST)
                      + b_ref[...])
    return pl.pallas_call(
        body, out_shape=jax.ShapeDtypeStruct((N_PAD, D), jnp.float32))(hs, cnt, wv, bv)


def _tc_he(hs, cnt, we, be):
    def body(hs_ref, c_ref, w_ref, b_ref, o_ref):
        de = jnp.maximum(jnp.sum(c_ref[...], axis=0), 1.0)[:, None]
        he = jnp.maximum((hs_ref[0] + hs_ref[1]) / de, 0.0)
        o_ref[...] = (jnp.dot(he, w_ref[...], preferred_element_type=jnp.float32,
                precision=jax.lax.Precision.HIGHEST)
                      + b_ref[...])
    return pl.pallas_call(
        body, out_shape=jax.ShapeDtypeStruct((M_PAD, D), jnp.float32))(hs, cnt, we, be)


def _tc_h3(hs, cnt):
    def body(hs_ref, c_ref, o_ref):
        dv = jnp.maximum(jnp.sum(c_ref[...], axis=0), 1.0)[:, None]
        h3 = jnp.maximum(_vjoin(hs_ref) / dv, 0.0)
        o_ref[pl.ds(0, N_PAD), :] = h3
        o_ref[pl.ds(N_PAD, H3_PAD - N_PAD), :] = jnp.zeros(
            (H3_PAD - N_PAD, D), jnp.float32)
    return pl.pallas_call(
        body, out_shape=jax.ShapeDtypeStruct((H3_PAD, D), jnp.float32))(hs, cnt)


def _tc_mlp(gs, w1, b1, w2, b2, w3, b3):
    def body(gs_ref, w1r, b1r, w2r, b2r, w3r, b3r, o_ref):
        g = jnp.sum(gs_ref[...], axis=0)[:G]
        g = jnp.maximum(jnp.dot(g, w1r[...], preferred_element_type=jnp.float32,
                precision=jax.lax.Precision.HIGHEST)
                        + b1r[...], 0.0)
        g = jnp.maximum(jnp.dot(g, w2r[...], preferred_element_type=jnp.float32,
                precision=jax.lax.Precision.HIGHEST)
                        + b2r[...], 0.0)
        o_ref[...] = (jnp.dot(g, w3r[...], preferred_element_type=jnp.float32,
                precision=jax.lax.Precision.HIGHEST)
                      + b3r[...])
    return pl.pallas_call(
        body, out_shape=jax.ShapeDtypeStruct((G, 1), jnp.float32))(
            gs, w1, b1, w2, b2, w3, b3)


def _range_sidx(tgt):
    c0 = jnp.where(tgt < NH, tgt, NH_PAD - 1)
    c1 = jnp.where(tgt >= NH, tgt - NH, NH_PAD - 1)
    return jnp.stack([c0, c1]).reshape(2, -1, BL)


def kernel(x, v_idx, e_idx, batch, atom_tables,
           layer_Wv, layer_bv, layer_We, layer_be,
           mlp_W1, mlp_b1, mlp_W2, mlp_b2, mlp_W3, mlp_b3):
    i32 = jnp.int32
    f32 = jnp.float32
    v_idx = v_idx.astype(i32)
    e_idx = e_idx.astype(i32)

    xt = jnp.concatenate(
        [x.astype(i32), jnp.zeros((N_PAD - N, F), i32)]).T

    pad_e = E_PAD - E
    vidx_g = jnp.concatenate([v_idx, jnp.zeros((pad_e,), i32)]).reshape(-1, BL)
    eidx_s = jnp.concatenate(
        [e_idx, jnp.full((pad_e,), M_PAD - 1, i32)]).reshape(-1, BL)
    eidx_g = jnp.concatenate([e_idx, jnp.zeros((pad_e,), i32)]).reshape(-1, BL)
    vidx_s = _range_sidx(jnp.concatenate(
        [v_idx, jnp.full((pad_e,), N_PAD - 1, i32)]))

    deg_rows = M_PAD + N_PAD
    deg_t = jnp.concatenate(
        [e_idx, M_PAD + v_idx,
         jnp.full((DEG_PAD - 2 * E,), deg_rows - 1, i32)])
    deg_g = (deg_t % BL).reshape(-1, BL)
    deg_tile = (jnp.arange(DEG_PAD, dtype=i32)
                // ((DEG_PAD // NW // BL) * BL)) % NS
    deg_s = (deg_t // BL + BL * deg_tile).reshape(-1, BL)
    ident = jnp.eye(BL, dtype=f32)

    pool_g = jnp.concatenate(
        [jnp.arange(N_PAD, dtype=i32),
         jnp.zeros((POOL_PAD - N_PAD,), i32)]).reshape(-1, BL)
    pool_t = jnp.concatenate(
        [batch.astype(i32), jnp.full((N_PAD - N,), G_PAD - 1, i32),
         jnp.full((POOL_PAD - N_PAD,), G_PAD - 1, i32)])
    pool_s = (pool_t + G_PAD * (jnp.arange(POOL_PAD, dtype=i32) % 8)
              ).reshape(-1, BL)

    zeros128 = jnp.zeros((NH_PAD // NS, D), f32)

    bv = layer_bv.reshape(3, 1, D)
    be = layer_be.reshape(3, 1, D)
    b1 = mlp_b1.reshape(1, D)
    b2 = mlp_b2.reshape(1, D)
    b3 = mlp_b3.reshape(1, 1)

    deg = _sc_pair_split(DEG_PAD // NW // BL, NS * BL)(
        deg_g, deg_s, ident, zeros128[:BL])
    dcnt = deg.reshape(NC * NS, BL * BL)
    cnt_e = dcnt[:, :M_PAD]
    cnt_v = dcnt[:, M_PAD:deg_rows]

    hv = _tc_enc_hv(xt, atom_tables, layer_Wv[0], bv[0])
    hs = None
    for l in range(3):
        hes = _sc_pair_split(E_PAD // NW // BL, M_PAD)(
            vidx_g, eidx_s, hv, zeros128[:M_PAD // NS])
        he = _tc_he(hes, cnt_e, layer_We[l], be[l])
        hs = _sc_range_split(E_PAD // NS // BL, M_PAD)(
            eidx_g, vidx_s, he, zeros128)
        if l < 2:
            hv = _tc_hv(hs, cnt_v, layer_Wv[l + 1], bv[l + 1])

    h3 = _tc_h3(hs, cnt_v)
    gs = _sc_pair_split(POOL_PAD // NW // BL, 8 * G_PAD)(
        pool_g, pool_s, h3, zeros128[:8 * G_PAD // NS])
    out = _tc_mlp(gs.reshape(NC * 8, G_PAD, D), mlp_W1, b1, mlp_W2, b2,
                  mlp_W3, b3)
    return out.reshape(-1)


# speedup vs baseline: 1.7808x; 1.1501x over previous
import functools

import jax
import jax.numpy as jnp
from jax import lax
from jax.experimental import pallas as pl
from jax.experimental.pallas import tpu as pltpu
from jax.experimental.pallas import tpu_sc as plsc

NC, NS = 2, 16
NW = NC * NS
BL = 128

N = 10000
E = 320000
M = 2500
G = 256
D = 128
F = 9
V = 128

N_PAD = 10240
NH = N_PAD // 2
NH_PAD = 5248
M_PAD = 2560
G_PAD = 384
H3_PAD = 12288
E_PAD = 327680
DEG_PAD = 655360
POOL_PAD = 32768


def _mesh():
    return plsc.VectorSubcoreMesh(
        core_axis_name="c", subcore_axis_name="s",
        num_cores=NC, num_subcores=NS)


@functools.lru_cache(None)
def _sc_pair_split(rpt: int, a_rows: int):
    ar16 = a_rows // NS

    def body(gidx, sidx, table, zeros, out, gv, sv, b0, b1, acc, sem0, sem1):
        c = lax.axis_index("c")
        s = lax.axis_index("s")
        row0 = (c * NS + s) * rpt
        pltpu.sync_copy(zeros, acc.at[pl.ds(s * ar16, ar16)])
        pltpu.sync_copy(gidx.at[pl.ds(row0, rpt)], gv)
        pltpu.sync_copy(sidx.at[pl.ds(row0, rpt)], sv)
        plsc.subcore_barrier()

        def start(j, buf, sem):
            pltpu.async_copy(table.at[gv.at[j]], buf, sem)

        def finish(j, buf, sem):
            pltpu.make_async_copy(table.at[gv.at[j]], buf, sem).wait()
            pltpu.sync_copy(buf, acc.at[sv.at[j]], add=True)

        start(0, b0, sem0)
        start(1, b1, sem1)

        def step(i, carry):
            j0 = 2 * i

            finish(j0, b0, sem0)

            @pl.when(j0 + 2 < rpt)
            def _():
                start(j0 + 2, b0, sem0)

            finish(j0 + 1, b1, sem1)

            @pl.when(j0 + 3 < rpt)
            def _():
                start(j0 + 3, b1, sem1)

            return carry

        lax.fori_loop(0, rpt // 2, step, 0)
        plsc.subcore_barrier()
        pltpu.sync_copy(acc.at[pl.ds(s * ar16, ar16)],
                        out.at[c, pl.ds(s * ar16, ar16)])

    return pl.kernel(
        body,
        out_type=jax.ShapeDtypeStruct((NC, a_rows, D), jnp.float32),
        mesh=_mesh(),
        scratch_types=[
            pltpu.VMEM((rpt, BL), jnp.int32),
            pltpu.VMEM((rpt, BL), jnp.int32),
            pltpu.VMEM((BL, D), jnp.float32),
            pltpu.VMEM((BL, D), jnp.float32),
            pltpu.VMEM_SHARED((a_rows, D), jnp.float32),
            pltpu.SemaphoreType.DMA,
            pltpu.SemaphoreType.DMA,
        ],
    )


@functools.lru_cache(None)
def _sc_range_split(rpt: int, t_rows: int):
    ar16 = NH_PAD // NS

    def body(gidx, sidx, table, zeros, out, gv, sv, b0, b1, acc, sem0, sem1):
        c = lax.axis_index("c")
        s = lax.axis_index("s")
        row0 = s * rpt
        pltpu.sync_copy(zeros, acc.at[pl.ds(s * ar16, ar16)])
        pltpu.sync_copy(gidx.at[pl.ds(row0, rpt)], gv)
        pltpu.sync_copy(sidx.at[c, pl.ds(row0, rpt)], sv)
        plsc.subcore_barrier()

        def start(j, buf, sem):
            pltpu.async_copy(table.at[gv.at[j]], buf, sem)

        def finish(j, buf, sem):
            pltpu.make_async_copy(table.at[gv.at[j]], buf, sem).wait()
            pltpu.sync_copy(buf, acc.at[sv.at[j]], add=True)

        start(0, b0, sem0)
        start(1, b1, sem1)

        def step(i, carry):
            j0 = 2 * i

            finish(j0, b0, sem0)

            @pl.when(j0 + 2 < rpt)
            def _():
                start(j0 + 2, b0, sem0)

            finish(j0 + 1, b1, sem1)

            @pl.when(j0 + 3 < rpt)
            def _():
                start(j0 + 3, b1, sem1)

            return carry

        lax.fori_loop(0, rpt // 2, step, 0)
        plsc.subcore_barrier()
        pltpu.sync_copy(acc.at[pl.ds(s * ar16, ar16)],
                        out.at[c, pl.ds(s * ar16, ar16)])

    return pl.kernel(
        body,
        out_type=jax.ShapeDtypeStruct((NC, NH_PAD, D), jnp.float32),
        mesh=_mesh(),
        scratch_types=[
            pltpu.VMEM((rpt, BL), jnp.int32),
            pltpu.VMEM((rpt, BL), jnp.int32),
            pltpu.VMEM((BL, D), jnp.float32),
            pltpu.VMEM((BL, D), jnp.float32),
            pltpu.VMEM_SHARED((NH_PAD, D), jnp.float32),
            pltpu.SemaphoreType.DMA,
            pltpu.SemaphoreType.DMA,
        ],
    )


def _vjoin(hs_ref):
    return jnp.concatenate([hs_ref[0, :NH], hs_ref[1, :NH]], axis=0)


def _tc_enc_hv(xt, at, wv, bv):
    def body(x_ref, at_ref, w_ref, b_ref, o_ref):
        h = jnp.zeros((N_PAD, D), jnp.float32)
        for f in range(F):
            oh = (jax.lax.broadcasted_iota(jnp.int32, (V, N_PAD), 0)
                  == x_ref[f][None, :]).astype(jnp.float32)
            h = h + jax.lax.dot_general(
                oh, at_ref[f], (((0,), (0,)), ((), ())),
                preferred_element_type=jnp.float32)
        o_ref[...] = (jnp.dot(h, w_ref[...], preferred_element_type=jnp.float32,
                precision=jax.lax.Precision.HIGHEST)
                      + b_ref[...])
    return pl.pallas_call(
        body, out_shape=jax.ShapeDtypeStruct((N_PAD, D), jnp.float32))(
            xt, at, wv, bv)


def _tc_hv(hs, cnt, wv, bv):
    def body(hs_ref, c_ref, w_ref, b_ref, o_ref):
        dv = jnp.maximum(jnp.sum(c_ref[...], axis=0), 1.0)[:, None]
        h = jnp.maximum(_vjoin(hs_ref) / dv, 0.0)
        o_ref[...] = (jnp.dot(h, w_ref[...], preferred_element_type=jnp.float32,
                precision=jax.lax.Precision.HIGHEST)
                      + b_ref[...])
    return pl.pallas_call(
        body, out_shape=jax.ShapeDtypeStruct((N_PAD, D), jnp.float32))(hs, cnt, wv, bv)


def _tc_he(hs, cnt, we, be):
    def body(hs_ref, c_ref, w_ref, b_ref, o_ref):
        de = jnp.maximum(jnp.sum(c_ref[...], axis=0), 1.0)[:, None]
        he = jnp.maximum((hs_ref[0] + hs_ref[1]) / de, 0.0)
        o_ref[...] = (jnp.dot(he, w_ref[...], preferred_element_type=jnp.float32,
                precision=jax.lax.Precision.HIGHEST)
                      + b_ref[...])
    return pl.pallas_call(
        body, out_shape=jax.ShapeDtypeStruct((M_PAD, D), jnp.float32))(hs, cnt, we, be)


def _tc_h3(hs, cnt):
    def body(hs_ref, c_ref, o_ref):
        dv = jnp.maximum(jnp.sum(c_ref[...], axis=0), 1.0)[:, None]
        h3 = jnp.maximum(_vjoin(hs_ref) / dv, 0.0)
        o_ref[pl.ds(0, N_PAD), :] = h3
        o_ref[pl.ds(N_PAD, H3_PAD - N_PAD), :] = jnp.zeros(
            (H3_PAD - N_PAD, D), jnp.float32)
    return pl.pallas_call(
        body, out_shape=jax.ShapeDtypeStruct((H3_PAD, D), jnp.float32))(hs, cnt)


def _tc_mlp(gs, w1, b1, w2, b2, w3, b3):
    def body(gs_ref, w1r, b1r, w2r, b2r, w3r, b3r, o_ref):
        g = jnp.sum(gs_ref[...], axis=0)[:G]
        g = jnp.maximum(jnp.dot(g, w1r[...], preferred_element_type=jnp.float32,
                precision=jax.lax.Precision.HIGHEST)
                        + b1r[...], 0.0)
        g = jnp.maximum(jnp.dot(g, w2r[...], preferred_element_type=jnp.float32,
                precision=jax.lax.Precision.HIGHEST)
                        + b2r[...], 0.0)
        o_ref[...] = (jnp.dot(g, w3r[...], preferred_element_type=jnp.float32,
                precision=jax.lax.Precision.HIGHEST)
                      + b3r[...])
    return pl.pallas_call(
        body, out_shape=jax.ShapeDtypeStruct((G, 1), jnp.float32))(
            gs, w1, b1, w2, b2, w3, b3)


def _range_sidx(tgt):
    c0 = jnp.where(tgt < NH, tgt, NH_PAD - 1)
    c1 = jnp.where(tgt >= NH, tgt - NH, NH_PAD - 1)
    return jnp.stack([c0, c1]).reshape(2, -1, BL)


def kernel(x, v_idx, e_idx, batch, atom_tables,
           layer_Wv, layer_bv, layer_We, layer_be,
           mlp_W1, mlp_b1, mlp_W2, mlp_b2, mlp_W3, mlp_b3):
    i32 = jnp.int32
    f32 = jnp.float32
    v_idx = v_idx.astype(i32)
    e_idx = e_idx.astype(i32)

    xt = jnp.concatenate(
        [x.astype(i32), jnp.zeros((N_PAD - N, F), i32)]).T

    pad_e = E_PAD - E
    vidx_g = jnp.concatenate([v_idx, jnp.zeros((pad_e,), i32)]).reshape(-1, BL)
    eidx_s = jnp.concatenate(
        [e_idx, jnp.full((pad_e,), M_PAD - 1, i32)]).reshape(-1, BL)
    eidx_g = jnp.concatenate([e_idx, jnp.zeros((pad_e,), i32)]).reshape(-1, BL)
    vidx_s = _range_sidx(jnp.concatenate(
        [v_idx, jnp.full((pad_e,), N_PAD - 1, i32)]))

    deg_rows = M_PAD + N_PAD
    deg_t = jnp.concatenate(
        [e_idx, M_PAD + v_idx,
         jnp.full((DEG_PAD - 2 * E,), deg_rows - 1, i32)])
    deg_g = (deg_t % BL).reshape(-1, BL)
    deg_p = jnp.arange(DEG_PAD, dtype=i32)
    deg_tile = (deg_p // ((DEG_PAD // NW // BL) * BL)) % NS
    deg_base = deg_t // BL
    deg_row = jnp.where(deg_t < M_PAD, deg_base + 20 * (deg_p % 6),
                        deg_base + 100)
    deg_s = (deg_row + 256 * deg_tile).reshape(-1, BL)
    ident = jnp.eye(BL, dtype=f32)

    pad_p = jnp.arange(POOL_PAD - N_PAD, dtype=i32)
    pool_g = jnp.concatenate(
        [jnp.arange(N_PAD, dtype=i32),
         N_PAD + pad_p % (H3_PAD - N_PAD)]).reshape(-1, BL)
    pool_s = jnp.concatenate(
        [jnp.concatenate(
            [batch.astype(i32), jnp.full((N_PAD - N,), G_PAD - 1, i32)])
         + G_PAD * (jnp.arange(N_PAD, dtype=i32) % 8),
         pad_p % (8 * G_PAD)]).reshape(-1, BL)

    zeros128 = jnp.zeros((NH_PAD // NS, D), f32)

    bv = layer_bv.reshape(3, 1, D)
    be = layer_be.reshape(3, 1, D)
    b1 = mlp_b1.reshape(1, D)
    b2 = mlp_b2.reshape(1, D)
    b3 = mlp_b3.reshape(1, 1)

    deg = _sc_pair_split(DEG_PAD // NW // BL, NS * 256)(
        deg_g, deg_s, ident, zeros128[:256])
    dreg = deg.reshape(NC * NS, 256, BL)
    cnt_e = dreg[:, :120].reshape(NC * NS * 6, M_PAD)
    cnt_v = dreg[:, 120:200].reshape(NC * NS, N_PAD)

    hv = _tc_enc_hv(xt, atom_tables, layer_Wv[0], bv[0])
    hs = None
    for l in range(3):
        hes = _sc_pair_split(E_PAD // NW // BL, M_PAD)(
            vidx_g, eidx_s, hv, zeros128[:M_PAD // NS])
        he = _tc_he(hes, cnt_e, layer_We[l], be[l])
        hs = _sc_range_split(E_PAD // NS // BL, M_PAD)(
            eidx_g, vidx_s, he, zeros128)
        if l < 2:
            hv = _tc_hv(hs, cnt_v, layer_Wv[l + 1], bv[l + 1])

    h3 = _tc_h3(hs, cnt_v)
    gs = _sc_pair_split(POOL_PAD // NW // BL, 8 * G_PAD)(
        pool_g, pool_s, h3, zeros128[:8 * G_PAD // NS])
    out = _tc_mlp(gs.reshape(NC * 8, G_PAD, D), mlp_W1, b1, mlp_W2, b2,
                  mlp_W3, b3)
    return out.reshape(-1)
